# baseline (speedup 1.0000x reference)
import functools
import math

import jax
import jax.numpy as jnp
from jax import lax
from jax.experimental import pallas as pl
from jax.experimental.pallas import tpu as pltpu
from jax.experimental.pallas import tpu_sc as plsc

D = 64
SCALE = math.sqrt(D)
V = 1000000

_info = plsc.get_sparse_core_info()
NC, NS, L = _info.num_cores, _info.num_subcores, _info.num_lanes
NW = NC * NS

NBLK = (V // 128) * 1
TAIL0 = NBLK * 128


def _iota16():
    return lax.iota(jnp.int32, L)


def _splat(v):
    return jnp.full((L,), v, jnp.int32)


@jax.jit
def _table_prep(lut_t, tail_t):
    mesh = plsc.VectorSubcoreMesh(core_axis_name="c", subcore_axis_name="s")

    @functools.partial(
        pl.kernel,
        mesh=mesh,
        compiler_params=pltpu.CompilerParams(needs_layout_passes=False),
        out_type=jax.ShapeDtypeStruct((V // 2, 128), jnp.float32),
        scratch_types=[
            pltpu.VMEM((2, D, 128), jnp.float32),
            pltpu.VMEM((2, D, 128), jnp.float32),
            pltpu.VMEM((D, D), jnp.float32),
            pltpu.VMEM((32, 128), jnp.float32),
        ]
        + [pltpu.SemaphoreType.DMA] * 4,
    )
    def body(src, tail, dst, vbuf, obuf, tin, tout, *sems):
        isems, osems = sems[:2], sems[2:]
        wid = lax.axis_index("s") * NC + lax.axis_index("c")

        def start_in(t, b):
            c0 = (wid + NW * t) * 128
            pltpu.make_async_copy(
                src.at[:, pl.ds(c0, 128)], vbuf.at[b], isems[b]
            ).start()

        def wait_in(t, b):
            c0 = (wid + NW * t) * 128
            pltpu.make_async_copy(
                src.at[:, pl.ds(c0, 128)], vbuf.at[b], isems[b]
            ).wait()

        def start_out(t, b):
            r0 = (wid + NW * t) * 64
            pltpu.make_async_copy(
                obuf.at[b], dst.at[pl.ds(r0, 64)], osems[b]
            ).start()

        def wait_out(t, b):
            r0 = (wid + NW * t) * 64
            pltpu.make_async_copy(
                obuf.at[b], dst.at[pl.ds(r0, 64)], osems[b]
            ).wait()

        def transpose_block(b):
            iot = _iota16()

            def prow(p, _):
                for v in range(8):
                    h = v // 4
                    vals = plsc.load_gather(
                        vbuf.at[b], [iot + 16 * (v % 4), _splat(2 * p + h)]
                    )
                    obuf[b, p, pl.ds(16 * v, L)] = vals * SCALE
                return 0

            lax.fori_loop(0, D, prow, 0, unroll=2)

        for b in range(2):
            @pl.when(wid + NW * b < NBLK)
            def _():
                start_in(b, b)

        def group(tt, _):
            for b in range(2):
                t = 2 * tt + b
                blk = wid + NW * t

                @pl.when(blk < NBLK)
                def _():
                    wait_in(t, b)

                    @pl.when(t >= 2)
                    def _():
                        wait_out(t - 2, b)

                    transpose_block(b)
                    start_out(t, b)

                    @pl.when(wid + NW * (t + 2) < NBLK)
                    def _():
                        start_in(t + 2, b)

            return 0

        lax.fori_loop(0, 123, group, 0)

        nblk_w = (NBLK - wid + NW - 1) // NW
        for b in range(2):
            @pl.when(nblk_w > b)
            def _():
                t_b = ((nblk_w - 1 - b) // 2) * 2 + b
                wait_out(t_b, b)

        @pl.when(wid == 0)
        def _():
            pltpu.sync_copy(tail, tin)
            iot = _iota16()

            def prow(p, _):
                for v in range(8):
                    h = v // 4
                    vals = plsc.load_gather(
                        tin, [iot + 16 * (v % 4), _splat(2 * p + h)]
                    )
                    tout[p, pl.ds(16 * v, L)] = vals * SCALE
                return 0

            lax.fori_loop(0, 32, prow, 0)
            pltpu.sync_copy(tout, dst.at[pl.ds(TAIL0 // 2, 32)])

    return body(lut_t, tail_t)


NB = 4
TPW = 200


@jax.jit
def _gather_tr(tbl, xv):
    mesh = plsc.VectorSubcoreMesh(core_axis_name="c", subcore_axis_name="s")

    @functools.partial(
        pl.kernel,
        mesh=mesh,
        compiler_params=pltpu.CompilerParams(
            use_tc_tiling_on_sc=False, needs_layout_passes=False
        ),
        out_type=jax.ShapeDtypeStruct((200, 8, 32, 8, 128), jnp.float32),
        scratch_types=[
            pltpu.VMEM((25, 8, 128), jnp.int32),
            pltpu.VMEM((NB, 128, D), jnp.float32),
            pltpu.VMEM((NB, 8, 8, 128), jnp.float32),
        ]
        + [pltpu.SemaphoreType.DMA] * (2 * NB),
    )
    def body(tbl_hbm, xv_hbm, out_hbm, idx_v, rows_v, tbuf, *sems):
        gsems, wsems = sems[:NB], sems[NB:]
        wid = lax.axis_index("s") * NC + lax.axis_index("c")

        pltpu.sync_copy(xv_hbm.at[pl.ds(wid * 25, 25)], idx_v)

        def task_coords(t):
            lg, jl = t // 8, t % 8
            g = wid * 25 + lg
            jh, ih = g // 32, g % 32
            return lg, jl, jh * 8 + jl, ih

        def start_gather(t, b):
            lg, jl, _, _ = task_coords(t)
            pltpu.make_async_copy(
                tbl_hbm.at[idx_v.at[lg, jl]], rows_v.at[b], gsems[b]
            ).start()

        def wait_gather(t, b):
            lg, jl, _, _ = task_coords(t)
            pltpu.make_async_copy(
                tbl_hbm.at[idx_v.at[lg, jl]], rows_v.at[b], gsems[b]
            ).wait()

        def start_write(t, b):
            _, _, j, ih = task_coords(t)
            for kh in range(8):
                pltpu.make_async_copy(
                    tbuf.at[b, kh], out_hbm.at[j, kh, ih], wsems[b]
                ).start()

        def wait_write(t, b):
            _, _, j, ih = task_coords(t)
            for kh in range(8):
                pltpu.make_async_copy(
                    tbuf.at[b, kh], out_hbm.at[j, kh, ih], wsems[b]
                ).wait()

        def transpose_task(b):
            iot = _iota16()

            def prow(k, _):
                kh, kl = k // 8, k % 8
                for u in range(D // L):
                    vals = plsc.load_gather(
                        rows_v.at[b], [iot + 16 * u, _splat(k)]
                    )
                    tbuf[b, kh, kl, pl.ds(16 * u, L)] = vals
                return 0

            lax.fori_loop(0, D, prow, 0, unroll=2)

        for b in range(NB):
            start_gather(b, b)

        def group(tt, _):
            for b in range(NB):
                t = NB * tt + b
                wait_gather(t, b)

                @pl.when(t >= NB)
                def _():
                    wait_write(t - NB, b)

                transpose_task(b)
                start_write(t, b)

                @pl.when(t + NB < TPW)
                def _():
                    start_gather(t + NB, b)

            return 0

        lax.fori_loop(0, TPW // NB, group, 0)

        for b in range(NB):
            wait_write(TPW - NB + b, b)

    return body(tbl, xv)


@jax.jit
def _gather_plain(idx2d, tbl):
    nch = 200
    mesh = plsc.VectorSubcoreMesh(core_axis_name="c", subcore_axis_name="s")

    @functools.partial(
        pl.kernel,
        mesh=mesh,
        compiler_params=pltpu.CompilerParams(
            use_tc_tiling_on_sc=False, needs_layout_passes=False
        ),
        out_type=jax.ShapeDtypeStruct((819200, D), jnp.float32),
        scratch_types=[
            pltpu.VMEM((nch, 128), jnp.int32),
            pltpu.VMEM((NB, 128, D), jnp.float32),
        ]
        + [pltpu.SemaphoreType.DMA] * (2 * NB),
    )
    def body(idx_hbm, tbl_hbm, out_hbm, idx_v, rows_v, *sems):
        gsems, wsems = sems[:NB], sems[NB:]
        wid = lax.axis_index("s") * NC + lax.axis_index("c")
        row0 = wid * (nch * 128)
        pltpu.sync_copy(idx_hbm.at[pl.ds(wid * nch, nch)], idx_v)

        def sg(chunk, b):
            pltpu.make_async_copy(
                tbl_hbm.at[idx_v.at[chunk]], rows_v.at[b], gsems[b]
            ).start()

        def wg(chunk, b):
            pltpu.make_async_copy(
                tbl_hbm.at[idx_v.at[chunk]], rows_v.at[b], gsems[b]
            ).wait()

        def sw(chunk, b):
            pltpu.make_async_copy(
                rows_v.at[b], out_hbm.at[pl.ds(row0 + chunk * 128, 128)], wsems[b]
            ).start()

        def ww(chunk, b):
            pltpu.make_async_copy(
                rows_v.at[b], out_hbm.at[pl.ds(row0 + chunk * 128, 128)], wsems[b]
            ).wait()

        for b in range(NB):
            sg(b, b)

        def group(t, _):
            for b in range(NB):
                g = t * NB + b
                wg(g, b)
                sw(g, b)

                @pl.when(g + NB < nch)
                def _():
                    ww(g, b)
                    sg(g + NB, b)

            return 0

        lax.fori_loop(0, nch // NB, group, 0)
        for b in range(NB):
            ww(nch - NB + b, b)

    return body(idx2d, tbl)


def kernel(x, lut):
    xi = x.astype(jnp.int32)
    tbl2 = _table_prep(lut.T, lut[TAIL0:].T)
    tbl = tbl2.reshape(V, D)
    out = _gather_plain(xi.reshape(6400, 128), tbl)
    return out.reshape(4096, 200, D)


# speedup vs baseline: 1.5876x; 1.5876x over previous
import functools
import math

import jax
import jax.numpy as jnp
from jax import lax
from jax.experimental import pallas as pl
from jax.experimental.pallas import tpu as pltpu
from jax.experimental.pallas import tpu_sc as plsc

D = 64
SCALE = math.sqrt(D)
V = 1000000

_info = plsc.get_sparse_core_info()
NC, NS, L = _info.num_cores, _info.num_subcores, _info.num_lanes
NW = NC * NS

NBLK = (V // 128) * 1
TAIL0 = NBLK * 128


def _iota16():
    return lax.iota(jnp.int32, L)


def _splat(v):
    return jnp.full((L,), v, jnp.int32)


@jax.jit
def _table_prep(lut_t, tail_t):
    mesh = plsc.VectorSubcoreMesh(core_axis_name="c", subcore_axis_name="s")

    @functools.partial(
        pl.kernel,
        mesh=mesh,
        compiler_params=pltpu.CompilerParams(needs_layout_passes=False),
        out_type=jax.ShapeDtypeStruct((V // 2, 128), jnp.float32),
        scratch_types=[
            pltpu.VMEM((2, D, 128), jnp.float32),
            pltpu.VMEM((2, D, 128), jnp.float32),
            pltpu.VMEM((D, D), jnp.float32),
            pltpu.VMEM((32, 128), jnp.float32),
        ]
        + [pltpu.SemaphoreType.DMA] * 4,
    )
    def body(src, tail, dst, vbuf, obuf, tin, tout, *sems):
        isems, osems = sems[:2], sems[2:]
        wid = lax.axis_index("s") * NC + lax.axis_index("c")

        def start_in(t, b):
            c0 = (wid + NW * t) * 128
            pltpu.make_async_copy(
                src.at[:, pl.ds(c0, 128)], vbuf.at[b], isems[b]
            ).start()

        def wait_in(t, b):
            c0 = (wid + NW * t) * 128
            pltpu.make_async_copy(
                src.at[:, pl.ds(c0, 128)], vbuf.at[b], isems[b]
            ).wait()

        def start_out(t, b):
            r0 = (wid + NW * t) * 64
            pltpu.make_async_copy(
                obuf.at[b], dst.at[pl.ds(r0, 64)], osems[b]
            ).start()

        def wait_out(t, b):
            r0 = (wid + NW * t) * 64
            pltpu.make_async_copy(
                obuf.at[b], dst.at[pl.ds(r0, 64)], osems[b]
            ).wait()

        ridx = [_iota16() + 16 * m for m in range(4)]

        def transpose_block(b):
            @plsc.parallel_loop(0, D, unroll=4)
            def _(p):
                c0 = _splat(2 * p)
                c1 = c0 + 1
                for v in range(8):
                    vals = plsc.load_gather(
                        vbuf.at[b], [ridx[v % 4], c1 if v >= 4 else c0]
                    )
                    obuf[b, p, pl.ds(16 * v, L)] = vals * SCALE

        for b in range(2):
            @pl.when(wid + NW * b < NBLK)
            def _():
                start_in(b, b)

        def group(tt, _):
            for b in range(2):
                t = 2 * tt + b
                blk = wid + NW * t

                @pl.when(blk < NBLK)
                def _():
                    wait_in(t, b)

                    @pl.when(t >= 2)
                    def _():
                        wait_out(t - 2, b)

                    transpose_block(b)
                    start_out(t, b)

                    @pl.when(wid + NW * (t + 2) < NBLK)
                    def _():
                        start_in(t + 2, b)

            return 0

        lax.fori_loop(0, 123, group, 0)

        nblk_w = (NBLK - wid + NW - 1) // NW
        for b in range(2):
            @pl.when(nblk_w > b)
            def _():
                t_b = ((nblk_w - 1 - b) // 2) * 2 + b
                wait_out(t_b, b)

        @pl.when(wid == 0)
        def _():
            pltpu.sync_copy(tail, tin)
            iot = _iota16()

            def prow(p, _):
                for v in range(8):
                    h = v // 4
                    vals = plsc.load_gather(
                        tin, [iot + 16 * (v % 4), _splat(2 * p + h)]
                    )
                    tout[p, pl.ds(16 * v, L)] = vals * SCALE
                return 0

            lax.fori_loop(0, 32, prow, 0)
            pltpu.sync_copy(tout, dst.at[pl.ds(TAIL0 // 2, 32)])

    return body(lut_t, tail_t)


NB = 4
TPW = 200


@jax.jit
def _gather_tr(tbl, xv):
    mesh = plsc.VectorSubcoreMesh(core_axis_name="c", subcore_axis_name="s")

    @functools.partial(
        pl.kernel,
        mesh=mesh,
        compiler_params=pltpu.CompilerParams(
            use_tc_tiling_on_sc=False, needs_layout_passes=False
        ),
        out_type=jax.ShapeDtypeStruct((200, 8, 32, 8, 128), jnp.float32),
        scratch_types=[
            pltpu.VMEM((25, 8, 128), jnp.int32),
            pltpu.VMEM((NB, 128, D), jnp.float32),
            pltpu.VMEM((NB, 8, 8, 128), jnp.float32),
        ]
        + [pltpu.SemaphoreType.DMA] * (2 * NB),
    )
    def body(tbl_hbm, xv_hbm, out_hbm, idx_v, rows_v, tbuf, *sems):
        gsems, wsems = sems[:NB], sems[NB:]
        wid = lax.axis_index("s") * NC + lax.axis_index("c")

        pltpu.sync_copy(xv_hbm.at[pl.ds(wid * 25, 25)], idx_v)

        def task_coords(t):
            lg, jl = t // 8, t % 8
            g = wid * 25 + lg
            jh, ih = g // 32, g % 32
            return lg, jl, jh * 8 + jl, ih

        def start_gather(t, b):
            lg, jl, _, _ = task_coords(t)
            pltpu.make_async_copy(
                tbl_hbm.at[idx_v.at[lg, jl]], rows_v.at[b], gsems[b]
            ).start()

        def wait_gather(t, b):
            lg, jl, _, _ = task_coords(t)
            pltpu.make_async_copy(
                tbl_hbm.at[idx_v.at[lg, jl]], rows_v.at[b], gsems[b]
            ).wait()

        def start_write(t, b):
            _, _, j, ih = task_coords(t)
            for kh in range(8):
                pltpu.make_async_copy(
                    tbuf.at[b, kh], out_hbm.at[j, kh, ih], wsems[b]
                ).start()

        def wait_write(t, b):
            _, _, j, ih = task_coords(t)
            for kh in range(8):
                pltpu.make_async_copy(
                    tbuf.at[b, kh], out_hbm.at[j, kh, ih], wsems[b]
                ).wait()

        def transpose_task(b):
            iot = _iota16()

            def prow(k, _):
                kh, kl = k // 8, k % 8
                for u in range(D // L):
                    vals = plsc.load_gather(
                        rows_v.at[b], [iot + 16 * u, _splat(k)]
                    )
                    tbuf[b, kh, kl, pl.ds(16 * u, L)] = vals
                return 0

            lax.fori_loop(0, D, prow, 0, unroll=2)

        for b in range(NB):
            start_gather(b, b)

        def group(tt, _):
            for b in range(NB):
                t = NB * tt + b
                wait_gather(t, b)

                @pl.when(t >= NB)
                def _():
                    wait_write(t - NB, b)

                transpose_task(b)
                start_write(t, b)

                @pl.when(t + NB < TPW)
                def _():
                    start_gather(t + NB, b)

            return 0

        lax.fori_loop(0, TPW // NB, group, 0)

        for b in range(NB):
            wait_write(TPW - NB + b, b)

    return body(tbl, xv)


@jax.jit
def _gather_plain(idx2d, tbl):
    nch = 200
    mesh = plsc.VectorSubcoreMesh(core_axis_name="c", subcore_axis_name="s")

    @functools.partial(
        pl.kernel,
        mesh=mesh,
        compiler_params=pltpu.CompilerParams(
            use_tc_tiling_on_sc=False, needs_layout_passes=False
        ),
        out_type=jax.ShapeDtypeStruct((819200, D), jnp.float32),
        scratch_types=[
            pltpu.VMEM((nch, 128), jnp.int32),
            pltpu.VMEM((NB, 128, D), jnp.float32),
        ]
        + [pltpu.SemaphoreType.DMA] * (2 * NB),
    )
    def body(idx_hbm, tbl_hbm, out_hbm, idx_v, rows_v, *sems):
        gsems, wsems = sems[:NB], sems[NB:]
        wid = lax.axis_index("s") * NC + lax.axis_index("c")
        row0 = wid * (nch * 128)
        pltpu.sync_copy(idx_hbm.at[pl.ds(wid * nch, nch)], idx_v)

        def sg(chunk, b):
            pltpu.make_async_copy(
                tbl_hbm.at[idx_v.at[chunk]], rows_v.at[b], gsems[b]
            ).start()

        def wg(chunk, b):
            pltpu.make_async_copy(
                tbl_hbm.at[idx_v.at[chunk]], rows_v.at[b], gsems[b]
            ).wait()

        def sw(chunk, b):
            pltpu.make_async_copy(
                rows_v.at[b], out_hbm.at[pl.ds(row0 + chunk * 128, 128)], wsems[b]
            ).start()

        def ww(chunk, b):
            pltpu.make_async_copy(
                rows_v.at[b], out_hbm.at[pl.ds(row0 + chunk * 128, 128)], wsems[b]
            ).wait()

        for b in range(NB):
            sg(b, b)

        def group(t, _):
            for b in range(NB):
                g = t * NB + b
                wg(g, b)
                sw(g, b)

                @pl.when(g + NB < nch)
                def _():
                    ww(g, b)
                    sg(g + NB, b)

            return 0

        lax.fori_loop(0, nch // NB, group, 0)
        for b in range(NB):
            ww(nch - NB + b, b)

    return body(idx2d, tbl)


def kernel(x, lut):
    xi = x.astype(jnp.int32)
    tbl2 = _table_prep(lut.T, lut[TAIL0:].T)
    tbl = tbl2.reshape(V, D)
    out = _gather_plain(xi.reshape(6400, 128), tbl)
    return out.reshape(4096, 200, D)


# speedup vs baseline: 3.4799x; 2.1919x over previous
import functools
import math

import jax
import jax.numpy as jnp
from jax import lax
from jax.experimental import pallas as pl
from jax.experimental.pallas import tpu as pltpu
from jax.experimental.pallas import tpu_sc as plsc

D = 64
SCALE = math.sqrt(D)
V = 1000000

_info = plsc.get_sparse_core_info()
NC, NS, L = _info.num_cores, _info.num_subcores, _info.num_lanes
NW = NC * NS

NBLK = V // 128
TAIL0 = NBLK * 128


def _iota16():
    return lax.iota(jnp.int32, L)


def _splat(v):
    return jnp.full((L,), v, jnp.int32)


def _transpose_scaled(src2d, dst1d, rows, cols, scale, b=None):
    iot = _iota16()
    lead = [] if b is None else [_splat(b)]

    @plsc.parallel_loop(0, L, unroll=2)
    def _(d):
        dm = (iot + d) & (L - 1)
        sd = dm * rows + iot
        for kb in range(rows // L):
            k0 = kb * L
            ridx = iot + k0
            for cb in range(cols // L):
                c0 = cb * L
                vals = plsc.load_gather(src2d, lead + [ridx, dm + c0])
                if scale is not None:
                    vals = vals * scale
                plsc.store_scatter(dst1d, lead + [sd + (c0 * rows + k0)], vals)


@jax.jit
def _table_prep(lut_t, tail_t):
    mesh = plsc.VectorSubcoreMesh(core_axis_name="c", subcore_axis_name="s")

    @functools.partial(
        pl.kernel,
        mesh=mesh,
        compiler_params=pltpu.CompilerParams(needs_layout_passes=False),
        out_type=jax.ShapeDtypeStruct((V * D,), jnp.float32),
        scratch_types=[
            pltpu.VMEM((2, D, 128), jnp.float32),
            pltpu.VMEM((2, D * 128), jnp.float32),
            pltpu.VMEM((D, D), jnp.float32),
            pltpu.VMEM((D * D,), jnp.float32),
        ]
        + [pltpu.SemaphoreType.DMA] * 4,
    )
    def body(src, tail, dst, vbuf, obuf, tin, tout, *sems):
        isems, osems = sems[:2], sems[2:]
        wid = lax.axis_index("s") * NC + lax.axis_index("c")

        def start_in(t, b):
            c0 = (wid + NW * t) * 128
            pltpu.make_async_copy(
                src.at[:, pl.ds(c0, 128)], vbuf.at[b], isems[b]
            ).start()

        def wait_in(t, b):
            c0 = (wid + NW * t) * 128
            pltpu.make_async_copy(
                src.at[:, pl.ds(c0, 128)], vbuf.at[b], isems[b]
            ).wait()

        def start_out(t, b):
            e0 = (wid + NW * t) * (128 * D)
            pltpu.make_async_copy(
                obuf.at[b], dst.at[pl.ds(e0, 128 * D)], osems[b]
            ).start()

        def wait_out(t, b):
            e0 = (wid + NW * t) * (128 * D)
            pltpu.make_async_copy(
                obuf.at[b], dst.at[pl.ds(e0, 128 * D)], osems[b]
            ).wait()

        for b in range(2):
            @pl.when(wid + NW * b < NBLK)
            def _():
                start_in(b, b)

        def group(tt, _):
            for b in range(2):
                t = 2 * tt + b
                blk = wid + NW * t

                @pl.when(blk < NBLK)
                def _():
                    wait_in(t, b)

                    @pl.when(t >= 2)
                    def _():
                        wait_out(t - 2, b)

                    _transpose_scaled(vbuf, obuf, D, 128, SCALE, b=b)
                    start_out(t, b)

                    @pl.when(wid + NW * (t + 2) < NBLK)
                    def _():
                        start_in(t + 2, b)

            return 0

        lax.fori_loop(0, 123, group, 0)

        nblk_w = (NBLK - wid + NW - 1) // NW
        for b in range(2):
            @pl.when(nblk_w > b)
            def _():
                t_b = ((nblk_w - 1 - b) // 2) * 2 + b
                wait_out(t_b, b)

        @pl.when(wid == 0)
        def _():
            pltpu.sync_copy(tail, tin)
            _transpose_scaled(tin, tout, D, D, SCALE)
            pltpu.sync_copy(tout, dst.at[pl.ds(TAIL0 * D, D * D)])

    return body(lut_t, tail_t)


NB = 4
TPW = 200


@jax.jit
def _gather_tr(tbl, xv):
    mesh = plsc.VectorSubcoreMesh(core_axis_name="c", subcore_axis_name="s")

    @functools.partial(
        pl.kernel,
        mesh=mesh,
        compiler_params=pltpu.CompilerParams(
            use_tc_tiling_on_sc=False, needs_layout_passes=False
        ),
        out_type=jax.ShapeDtypeStruct((200 * 8 * 32 * 8 * 128,), jnp.float32),
        scratch_types=[
            pltpu.VMEM((25, 8, 128), jnp.int32),
            pltpu.VMEM((NB, 128, D), jnp.float32),
            pltpu.VMEM((NB, 128 * D), jnp.float32),
        ]
        + [pltpu.SemaphoreType.DMA] * (2 * NB),
    )
    def body(tbl_hbm, xv_hbm, out_hbm, idx_v, rows_v, tbuf, *sems):
        gsems, wsems = sems[:NB], sems[NB:]
        wid = lax.axis_index("s") * NC + lax.axis_index("c")

        pltpu.sync_copy(xv_hbm.at[pl.ds(wid * 25, 25)], idx_v)

        def task_coords(t):
            lg, jl = t // 8, t % 8
            g = wid * 25 + lg
            jh, ih = g // 32, g % 32
            return lg, jl, jh * 8 + jl, ih

        def start_gather(t, b):
            lg, jl, _, _ = task_coords(t)
            pltpu.make_async_copy(
                tbl_hbm.at[idx_v.at[lg, jl]], rows_v.at[b], gsems[b]
            ).start()

        def wait_gather(t, b):
            lg, jl, _, _ = task_coords(t)
            pltpu.make_async_copy(
                tbl_hbm.at[idx_v.at[lg, jl]], rows_v.at[b], gsems[b]
            ).wait()

        def start_write(t, b):
            _, _, j, ih = task_coords(t)
            for kh in range(8):
                off = ((j * 8 + kh) * 32 + ih) * 1024
                pltpu.make_async_copy(
                    tbuf.at[b, pl.ds(kh * 1024, 1024)],
                    out_hbm.at[pl.ds(off, 1024)],
                    wsems[b],
                ).start()

        def wait_write(t, b):
            _, _, j, ih = task_coords(t)
            for kh in range(8):
                off = ((j * 8 + kh) * 32 + ih) * 1024
                pltpu.make_async_copy(
                    tbuf.at[b, pl.ds(kh * 1024, 1024)],
                    out_hbm.at[pl.ds(off, 1024)],
                    wsems[b],
                ).wait()

        for b in range(NB):
            start_gather(b, b)

        def group(tt, _):
            for b in range(NB):
                t = NB * tt + b
                wait_gather(t, b)

                @pl.when(t >= NB)
                def _():
                    wait_write(t - NB, b)

                _transpose_scaled(rows_v, tbuf, 128, D, None, b=b)
                start_write(t, b)

                @pl.when(t + NB < TPW)
                def _():
                    start_gather(t + NB, b)

            return 0

        lax.fori_loop(0, TPW // NB, group, 0)

        for b in range(NB):
            wait_write(TPW - NB + b, b)

    return body(tbl, xv)


def kernel(x, lut):
    xi = x.astype(jnp.int32)
    tbl_flat = _table_prep(lut.T, lut[TAIL0:].T)
    tbl = tbl_flat.reshape(V, D)
    xv = xi.reshape(32, 128, 25, 8).transpose(2, 0, 3, 1).reshape(800, 8, 128)
    out5 = _gather_tr(tbl, xv).reshape(200, 8, 32, 8, 128)
    return out5.transpose(2, 4, 0, 1, 3).reshape(4096, 200, D)
